# baseline (speedup 1.0000x reference)
import functools

import jax
import jax.numpy as jnp
from jax import lax
from jax.experimental import pallas as pl
from jax.experimental.pallas import tpu as pltpu
from jax.experimental.pallas import tpu_sc as plsc

L = 16
NC = 2
NS = 16
NW = NC * NS
ROWS, COLS = 16384, 200
PROWS = ROWS // 4
ROWS_W = PROWS // NW
NCH = 4
CHUNK = ROWS_W // NCH
FULL_VECS = COLS // L
TAIL_WOFF = COLS - L
TAIL_DUP = FULL_VECS * L - TAIL_WOFF
DUP_BYTES = 4 * TAIL_DUP
PAIR_STRIDE = 256
TAB_SIZE = ((4 + PAIR_STRIDE * 4) + 1) * L

_mesh = plsc.VectorSubcoreMesh(core_axis_name="c", subcore_axis_name="s")


@functools.partial(
    pl.kernel,
    mesh=_mesh,
    compiler_params=pltpu.CompilerParams(needs_layout_passes=False),
    out_type=jax.ShapeDtypeStruct((NW, L), jnp.float32),
    scratch_types=[
        pltpu.VMEM((2, CHUNK, COLS), jnp.int32),
        pltpu.VMEM((32,), jnp.float32),
        pltpu.VMEM((TAB_SIZE,), jnp.float32),
        pltpu.VMEM((L,), jnp.float32),
        pltpu.SemaphoreType.DMA,
        pltpu.SemaphoreType.DMA,
    ],
)
def _lookup_sum(x_hbm, tflat_hbm, out_hbm, xbuf, tbuf, tab, accbuf, sem0, sem1):
    cid = lax.axis_index("c")
    sid = lax.axis_index("s")
    wid = sid * NC + cid
    row0 = wid * ROWS_W
    sems = (sem0, sem1)

    def chunk_copy(c, b):
        return pltpu.make_async_copy(
            x_hbm.at[pl.ds(row0 + c * CHUNK, CHUNK)], xbuf.at[b], sems[b]
        )

    chunk_copy(0, 0).start()
    pltpu.sync_copy(tflat_hbm, tbuf)

    rs = []
    for k in range(5):
        v = tbuf[pl.ds(4 * k, L)]
        rs.append(v[0] + v[1] + v[2] + v[3])

    for b1 in range(5):
        for b0 in range(5):
            val = rs[b0] + rs[b1]
            tab[pl.ds((b0 + PAIR_STRIDE * b1) * L, L)] = jnp.broadcast_to(val, (L,))

    lanes = lax.iota(jnp.int32, L)
    tail_keep = lanes >= TAIL_DUP
    zero = jnp.zeros((L,), jnp.int32)

    def make_body(buf):
        def one_vec(r, woff, a0, a1, tail):
            v = buf[r, pl.ds(woff, L)]
            if tail:
                v = jnp.where(tail_keep, v, zero)
            p0 = ((v & 0xFFFF) << 4) + lanes
            p1 = ((v >> 16) << 4) + lanes
            a0 = a0 + plsc.load_gather(tab, [p0])
            a1 = a1 + plsc.load_gather(tab, [p1])
            return a0, a1

        def body(r, accs):
            a0, a1, a2, a3 = accs
            for u in range(FULL_VECS):
                if u % 2 == 0:
                    a0, a1 = one_vec(r, u * L, a0, a1, False)
                else:
                    a2, a3 = one_vec(r, u * L, a2, a3, False)
            a0, a1 = one_vec(r, TAIL_WOFF, a0, a1, True)
            return (a0, a1, a2, a3)

        return body

    zf = jnp.zeros((L,), jnp.float32)
    accs = (zf, zf, zf, zf)
    for c in range(NCH):
        b = c % 2
        chunk_copy(c, b).wait()
        if c + 1 < NCH:
            chunk_copy(c + 1, 1 - b).start()
        accs = lax.fori_loop(0, CHUNK, make_body(xbuf.at[b]), accs)

    correction = (ROWS_W * DUP_BYTES / L) * rs[0]
    total = (accs[0] + accs[1]) + (accs[2] + accs[3])
    accbuf[...] = total - jnp.broadcast_to(correction, (L,))
    pltpu.sync_copy(accbuf, out_hbm.at[wid])


_PACK_GRID = 8
_PACK_BLK = PROWS // _PACK_GRID


def _pack_body(a_ref, b_ref, c_ref, d_ref, o_ref):
    o_ref[...] = (a_ref[...] | (b_ref[...] << 8) | (c_ref[...] << 16)
                  | (d_ref[...] << 24))


def _pack(x):
    in_specs = [
        pl.BlockSpec((_PACK_BLK, COLS), lambda i, k=k: (i + k * _PACK_GRID, 0))
        for k in range(4)
    ]
    return pl.pallas_call(
        _pack_body,
        grid=(_PACK_GRID,),
        in_specs=in_specs,
        out_specs=pl.BlockSpec((_PACK_BLK, COLS), lambda i: (i, 0)),
        out_shape=jax.ShapeDtypeStruct((PROWS, COLS), jnp.int32),
    )(x, x, x, x)


def kernel(x, table):
    xp = _pack(x)
    tflat = jnp.zeros((32,), jnp.float32).at[:20].set(table.reshape(-1))
    partials = _lookup_sum(xp, tflat)
    return partials.sum()


# speedup vs baseline: 1.4850x; 1.4850x over previous
import functools

import jax
import jax.numpy as jnp
from jax import lax
from jax.experimental import pallas as pl
from jax.experimental.pallas import tpu as pltpu
from jax.experimental.pallas import tpu_sc as plsc

L = 16
NC = 2
NS = 16
NW = NC * NS
ROWS, COLS = 16384, 200
TROWS, TCOLS = COLS, ROWS
COLS_W = TCOLS // NW
NCH = 4
CCH = COLS_W // NCH
VECS_CH = CCH // (4 * L)
PAIR_STRIDE = 256
TAB_SIZE = ((4 + PAIR_STRIDE * 4) + 1) * L

_mesh = plsc.VectorSubcoreMesh(core_axis_name="c", subcore_axis_name="s")


@functools.partial(
    pl.kernel,
    mesh=_mesh,
    compiler_params=pltpu.CompilerParams(needs_layout_passes=False),
    out_type=jax.ShapeDtypeStruct((NW, L), jnp.float32),
    scratch_types=[
        pltpu.VMEM((2, TROWS, CCH), jnp.int8),
        pltpu.VMEM((32,), jnp.float32),
        pltpu.VMEM((TAB_SIZE,), jnp.float32),
        pltpu.VMEM((L,), jnp.float32),
        pltpu.SemaphoreType.DMA,
        pltpu.SemaphoreType.DMA,
    ],
)
def _lookup_sum(x_hbm, tflat_hbm, out_hbm, xbuf, tbuf, tab, accbuf, sem0, sem1):
    cid = lax.axis_index("c")
    sid = lax.axis_index("s")
    wid = sid * NC + cid
    col0 = wid * COLS_W
    sems = (sem0, sem1)

    def chunk_copy(c, b):
        return pltpu.make_async_copy(
            x_hbm.at[:, pl.ds(col0 + c * CCH, CCH)], xbuf.at[b], sems[b]
        )

    chunk_copy(0, 0).start()
    pltpu.sync_copy(tflat_hbm, tbuf)

    rs = []
    for k in range(5):
        v = tbuf[pl.ds(4 * k, L)]
        rs.append(v[0] + v[1] + v[2] + v[3])

    for b1 in range(5):
        for b0 in range(5):
            val = rs[b0] + rs[b1]
            tab[pl.ds((b0 + PAIR_STRIDE * b1) * L, L)] = jnp.broadcast_to(val, (L,))

    lanes = lax.iota(jnp.int32, L)

    def make_body(buf):
        def one_vec(r, boff, a0, a1):
            bv = buf[r, pl.ds(boff, 4 * L)]
            v = plsc.bitcast(bv, jnp.int32)
            p0 = ((v & 0xFFFF) << 4) + lanes
            p1 = ((v >> 16) << 4) + lanes
            a0 = a0 + plsc.load_gather(tab, [p0])
            a1 = a1 + plsc.load_gather(tab, [p1])
            return a0, a1

        def body(r, accs):
            a0, a1, a2, a3 = accs
            a0, a1 = one_vec(r, 0, a0, a1)
            a2, a3 = one_vec(r, 4 * L, a2, a3)
            return (a0, a1, a2, a3)

        return body

    zf = jnp.zeros((L,), jnp.float32)
    accs = (zf, zf, zf, zf)
    for c in range(NCH):
        b = c % 2
        chunk_copy(c, b).wait()
        if c + 1 < NCH:
            chunk_copy(c + 1, 1 - b).start()
        accs = lax.fori_loop(0, TROWS, make_body(xbuf.at[b]), accs)

    total = (accs[0] + accs[1]) + (accs[2] + accs[3])
    accbuf[...] = total
    pltpu.sync_copy(accbuf, out_hbm.at[wid])


def kernel(x, table):
    xt = x.astype(jnp.int8).T
    tflat = jnp.zeros((32,), jnp.float32).at[:20].set(table.reshape(-1))
    partials = _lookup_sum(xt, tflat)
    return partials.sum()


# speedup vs baseline: 1.5157x; 1.0207x over previous
import functools

import jax
import jax.numpy as jnp
from jax import lax
from jax.experimental import pallas as pl
from jax.experimental.pallas import tpu as pltpu
from jax.experimental.pallas import tpu_sc as plsc

L = 16
NC = 2
NS = 16
NW = NC * NS
ROWS, COLS = 16384, 200
TROWS, TCOLS = COLS, ROWS
COLS_W = TCOLS // NW
NCH = 2
CCH = COLS_W // NCH
VECS_CH = CCH // (4 * L)
PAIR_STRIDE = 256
TAB_SIZE = ((4 + PAIR_STRIDE * 4) + 1) * L

_mesh = plsc.VectorSubcoreMesh(core_axis_name="c", subcore_axis_name="s")


@functools.partial(
    pl.kernel,
    mesh=_mesh,
    compiler_params=pltpu.CompilerParams(needs_layout_passes=False),
    out_type=jax.ShapeDtypeStruct((NW, L), jnp.float32),
    scratch_types=[
        pltpu.VMEM((2, TROWS, CCH), jnp.int8),
        pltpu.VMEM((32,), jnp.float32),
        pltpu.VMEM((TAB_SIZE,), jnp.float32),
        pltpu.VMEM((L,), jnp.float32),
        pltpu.SemaphoreType.DMA,
        pltpu.SemaphoreType.DMA,
    ],
)
def _lookup_sum(x_hbm, tflat_hbm, out_hbm, xbuf, tbuf, tab, accbuf, sem0, sem1):
    cid = lax.axis_index("c")
    sid = lax.axis_index("s")
    wid = sid * NC + cid
    col0 = wid * COLS_W
    sems = (sem0, sem1)

    def chunk_copy(c, b):
        return pltpu.make_async_copy(
            x_hbm.at[:, pl.ds(col0 + c * CCH, CCH)], xbuf.at[b], sems[b]
        )

    chunk_copy(0, 0).start()
    pltpu.sync_copy(tflat_hbm, tbuf)

    rs = []
    for k in range(5):
        v = tbuf[pl.ds(4 * k, L)]
        rs.append(v[0] + v[1] + v[2] + v[3])

    for b1 in range(5):
        for b0 in range(5):
            val = rs[b0] + rs[b1]
            tab[pl.ds((b0 + PAIR_STRIDE * b1) * L, L)] = jnp.broadcast_to(val, (L,))

    lanes = lax.iota(jnp.int32, L)

    def make_body(buf):
        def one_vec(r, boff, a0, a1):
            bv = buf[r, pl.ds(boff, 4 * L)]
            v = plsc.bitcast(bv, jnp.int32)
            p0 = ((v & 0xFFFF) << 4) + lanes
            p1 = ((v >> 16) << 4) + lanes
            a0 = a0 + plsc.load_gather(tab, [p0])
            a1 = a1 + plsc.load_gather(tab, [p1])
            return a0, a1

        def body(r, accs):
            a0, a1, a2, a3 = accs
            for u in range(VECS_CH):
                if u % 2 == 0:
                    a0, a1 = one_vec(r, u * 4 * L, a0, a1)
                else:
                    a2, a3 = one_vec(r, u * 4 * L, a2, a3)
            return (a0, a1, a2, a3)

        return body

    zf = jnp.zeros((L,), jnp.float32)
    accs = (zf, zf, zf, zf)
    for c in range(NCH):
        b = c % 2
        chunk_copy(c, b).wait()
        if c + 1 < NCH:
            chunk_copy(c + 1, 1 - b).start()
        accs = lax.fori_loop(0, TROWS, make_body(xbuf.at[b]), accs)

    total = (accs[0] + accs[1]) + (accs[2] + accs[3])
    accbuf[...] = total
    pltpu.sync_copy(accbuf, out_hbm.at[wid])


def kernel(x, table):
    xt = x.astype(jnp.int8).T
    tflat = jnp.zeros((32,), jnp.float32).at[:20].set(table.reshape(-1))
    partials = _lookup_sum(xt, tflat)
    return partials.sum()
